# Initial kernel scaffold; baseline (speedup 1.0000x reference)
import jax
import jax.numpy as jnp
from jax.experimental import pallas as pl


def kernel(h_p, t_p, r_p, h_n, t_n, r_n, entity_emb, relation_emb, ext_emb, W, b):
    raise NotImplementedError("write your pallas kernel here")


# speedup vs baseline: 1.0406x; 1.0406x over previous
import functools

import jax
import jax.numpy as jnp
from jax import lax
from jax.experimental import pallas as pl
from jax.experimental.pallas import tpu as pltpu
from jax.experimental.pallas import tpu_sc as plsc

BATCH = 16384
D = 32
TOT = 2 * BATCH
NC = 2
NS = 16
NW = NC * NS
BPW = TOT // NW
CHUNK = 128
NCHUNK = BPW // CHUNK
L = 16
GROUPS = BPW // L


def _sqrt16(x):
    i = lax.bitcast_convert_type(x, jnp.int32)
    seed = jnp.int32(0x5F3759DF) - lax.shift_right_logical(i, 1)
    y = lax.bitcast_convert_type(seed, jnp.float32)
    for _ in range(3):
        y = y * (jnp.float32(1.5) - jnp.float32(0.5) * x * y * y)
    return x * y


def _dist_body(hi, ti, ri, ent, rel, out,
               hidx_v, tidx_v, ridx_v, h_rows, r_rows, t_rows, out_v, sem):
    wid = lax.axis_index("s") * NC + lax.axis_index("c")

    pltpu.sync_copy(hi.at[wid], hidx_v)
    pltpu.sync_copy(ti.at[wid], tidx_v)
    pltpu.sync_copy(ri.at[wid], ridx_v)

    copies = []
    for j in range(NCHUNK):
        base = j * CHUNK
        copies.append(pltpu.async_copy(
            ent.at[hidx_v.at[j]], h_rows.at[pl.ds(base, CHUNK)], sem))
        copies.append(pltpu.async_copy(
            ent.at[tidx_v.at[j]], t_rows.at[pl.ds(base, CHUNK)], sem))
        copies.append(pltpu.async_copy(
            rel.at[ridx_v.at[j]], r_rows.at[pl.ds(base, CHUNK)], sem))
    for c in copies:
        c.wait()

    lane = lax.iota(jnp.int32, L)

    def group(g, carry):
        row0 = g * L
        acc = jnp.zeros((L,), jnp.float32)
        for k in range(L):
            rr = row0 + k
            h0 = h_rows[rr, pl.ds(0, L)]
            h1 = h_rows[rr, pl.ds(L, L)]
            r0 = r_rows[rr, pl.ds(0, L)]
            r1 = r_rows[rr, pl.ds(L, L)]
            t0 = t_rows[rr, pl.ds(0, L)]
            t1 = t_rows[rr, pl.ds(L, L)]
            d0 = h0 + r0 - t0
            d1 = h1 + r1 - t1
            s = jnp.sum(d0 * d0 + d1 * d1)
            acc = jnp.where(lane == k, s, acc)
        out_v[pl.ds(row0, L)] = _sqrt16(acc + jnp.float32(1e-12))
        return carry

    lax.fori_loop(0, GROUPS, group, 0)

    pltpu.sync_copy(out_v, out.at[pl.ds(wid * BPW, BPW)])


_mesh = plsc.VectorSubcoreMesh(core_axis_name="c", subcore_axis_name="s")

_dist_call = functools.partial(
    pl.kernel,
    mesh=_mesh,
    out_type=jax.ShapeDtypeStruct((TOT,), jnp.float32),
    scratch_types=[
        pltpu.VMEM((NCHUNK, CHUNK), jnp.int32),
        pltpu.VMEM((NCHUNK, CHUNK), jnp.int32),
        pltpu.VMEM((NCHUNK, CHUNK), jnp.int32),
        pltpu.VMEM((BPW, D), jnp.float32),
        pltpu.VMEM((BPW, D), jnp.float32),
        pltpu.VMEM((BPW, D), jnp.float32),
        pltpu.VMEM((BPW,), jnp.float32),
        pltpu.SemaphoreType.DMA,
    ],
    compiler_params=pltpu.CompilerParams(
        needs_layout_passes=False,
        use_tc_tiling_on_sc=False,
    ),
)(_dist_body)


def kernel(h_p, t_p, r_p, h_n, t_n, r_n, entity_emb, relation_emb, ext_emb, W, b):
    h = jnp.concatenate([h_p, h_n]).reshape(NW, NCHUNK, CHUNK)
    t = jnp.concatenate([t_p, t_n]).reshape(NW, NCHUNK, CHUNK)
    r = jnp.concatenate([r_p, r_n]).reshape(NW, NCHUNK, CHUNK)
    dis = _dist_call(h, t, r, entity_emb, relation_emb)
    return dis[:BATCH], dis[BATCH:]
